# Initial kernel scaffold; baseline (speedup 1.0000x reference)
import jax
import jax.numpy as jnp
from jax.experimental import pallas as pl


def kernel(batch_x, ts, x, last_update, edge_index, batch_vec, time_w, time_b, tag_w0, tag_w1, tag_b, rg_wk, rg_wq, rg_wv, rg_ws, rg_b, lp_w1, lp_b1, lp_w2, lp_b2):
    raise NotImplementedError("write your pallas kernel here")


# speedup vs baseline: 8.5531x; 8.5531x over previous
import functools

import jax
import jax.numpy as jnp
from jax import lax
from jax.experimental import pallas as pl
from jax.experimental.pallas import tpu as pltpu
from jax.experimental.pallas import tpu_sc as plsc

N = 10000
E = 320000
G = 512
DR = 64
TD = 32
MD = 32
C = 128

NC = 2
NS = 16
NW = NC * NS
EW = E // NW
CB = 80
NCHUNK = EW // CB
BR = 40
NBLK = N // BR
KMAX = (NBLK + NS - 1) // NS
DW = 128
RB = 1000
NRB = N // RB

_f32 = jnp.float32
_mesh = plsc.VectorSubcoreMesh(core_axis_name="c", subcore_axis_name="s")


def _feat_body(bx, ts, xm, tw, tb, w0, h_o, o0_o):
    rel = jnp.cos(ts[...] * tw[...] + tb[...])
    h = jnp.concatenate([bx[...], rel, xm[...]], axis=1)
    h_o[...] = h
    o0_o[...] = jnp.dot(h, w0[...], preferred_element_type=_f32)


_feat_call = pl.pallas_call(
    _feat_body,
    grid=(NRB,),
    in_specs=[
        pl.BlockSpec((RB, DR), lambda i: (i, 0)),
        pl.BlockSpec((RB, 1), lambda i: (i, 0)),
        pl.BlockSpec((RB, MD), lambda i: (i, 0)),
        pl.BlockSpec((1, TD), lambda i: (0, 0)),
        pl.BlockSpec((1, TD), lambda i: (0, 0)),
        pl.BlockSpec((C, C), lambda i: (0, 0)),
    ],
    out_specs=[
        pl.BlockSpec((RB, C), lambda i: (i, 0)),
        pl.BlockSpec((RB, C), lambda i: (i, 0)),
    ],
    out_shape=[
        jax.ShapeDtypeStruct((N, C), _f32),
        jax.ShapeDtypeStruct((N, C), _f32),
    ],
)


def _dis(deg2):
    deg = deg2[0, :, 0:1] + deg2[1, :, 0:1]
    return jnp.where(deg > 0, lax.rsqrt(jnp.maximum(deg, 1e-12)), 0.0)


def _g_body(h, deg2, g_o):
    g_o[...] = _dis(deg2[...]) * h[...]


_g_call = pl.pallas_call(
    _g_body,
    grid=(NRB,),
    in_specs=[
        pl.BlockSpec((RB, C), lambda i: (i, 0)),
        pl.BlockSpec((NC, RB, DW), lambda i: (0, i, 0)),
    ],
    out_specs=pl.BlockSpec((RB, C), lambda i: (i, 0)),
    out_shape=jax.ShapeDtypeStruct((N, C), _f32),
)


def _h1_body(o0, hp2, deg2, w1, tbias, wk, wq, wv, ws, rbias,
             nk_o, nq_o, v_o, sr_o):
    hp2v = hp2[...]
    hp = _dis(deg2[...]) * (hp2v[0] + hp2v[1])
    h1 = (o0[...] + jnp.dot(hp, w1[...], preferred_element_type=_f32)
          + tbias[...])
    nk_o[...] = -jnp.dot(h1, wk[...], preferred_element_type=_f32)
    nq_o[...] = -jnp.dot(h1, wq[...], preferred_element_type=_f32)
    v_o[...] = jnp.dot(h1, wv[...], preferred_element_type=_f32)
    sr_o[...] = jnp.dot(h1, ws[...], preferred_element_type=_f32) + rbias[...]


_h1_call = pl.pallas_call(
    _h1_body,
    grid=(NRB,),
    in_specs=[
        pl.BlockSpec((RB, C), lambda i: (i, 0)),
        pl.BlockSpec((NC, RB, C), lambda i: (0, i, 0)),
        pl.BlockSpec((NC, RB, DW), lambda i: (0, i, 0)),
        pl.BlockSpec((C, C), lambda i: (0, 0)),
        pl.BlockSpec((1, C), lambda i: (0, 0)),
        pl.BlockSpec((C, C), lambda i: (0, 0)),
        pl.BlockSpec((C, C), lambda i: (0, 0)),
        pl.BlockSpec((C, C), lambda i: (0, 0)),
        pl.BlockSpec((C, C), lambda i: (0, 0)),
        pl.BlockSpec((1, C), lambda i: (0, 0)),
    ],
    out_specs=[pl.BlockSpec((RB, C), lambda i: (i, 0)) for _ in range(4)],
    out_shape=[jax.ShapeDtypeStruct((N, C), _f32) for _ in range(4)],
)


def _roots_body(bv, r_o):
    jcol = lax.broadcasted_iota(jnp.int32, (G, 1), 0)

    def stepk(k, acc):
        row = bv[pl.ds(k, 1), :]
        cmp = (row < jcol).astype(_f32)
        return acc + jnp.sum(cmp, axis=1, keepdims=True)

    acc = lax.fori_loop(0, NRB, stepk, jnp.zeros((G, 1), _f32))
    r_o[...] = jnp.minimum(acc, float(N - 1)).astype(jnp.int32)


_roots_call = pl.pallas_call(
    _roots_body,
    in_specs=[pl.BlockSpec((NRB, RB), lambda: (0, 0))],
    out_specs=pl.BlockSpec((G, 1), lambda: (0, 0)),
    out_shape=jax.ShapeDtypeStruct((G, 1), jnp.int32),
)


def _mlp_body(xr3, w1, b1, w2, b2, o):
    xr3v = xr3[...]
    xr = xr3v[0] + xr3v[1] + xr3v[2]
    hid = jnp.maximum(jnp.dot(xr, w1[...], preferred_element_type=_f32)
                      + b1[...], 0.0)
    o[...] = jnp.dot(hid, w2[...], preferred_element_type=_f32) + b2[...]


_mlp_call = pl.pallas_call(
    _mlp_body,
    in_specs=[
        pl.BlockSpec((3, G, C), lambda: (0, 0, 0)),
        pl.BlockSpec((C, C), lambda: (0, 0)),
        pl.BlockSpec((1, C), lambda: (0, 0)),
        pl.BlockSpec((C, 1), lambda: (0, 0)),
        pl.BlockSpec((1, 1), lambda: (0, 0)),
    ],
    out_specs=pl.BlockSpec((G, 1), lambda: (0, 0)),
    out_shape=jax.ShapeDtypeStruct((G, 1), _f32),
)


def _zero_wide(zb, rows, width):
    def fill(i, _):
        for j in range(width // 16):
            zb[i, pl.ds(j * 16, 16)] = jnp.zeros((16,), _f32)
        return 0
    lax.fori_loop(0, rows, fill, 0)


def _acc_zero(zb, acc, s):
    def zcp(k, _):
        b = s + k * NS

        @pl.when(b < NBLK)
        def _():
            pltpu.sync_copy(zb, acc.at[pl.ds(b * BR, BR)])
        return 0
    lax.fori_loop(0, KMAX, zcp, 0)


def _acc_out(acc, out_hbm, c, s):
    def ocp(k, _):
        b = s + k * NS

        @pl.when(b < NBLK)
        def _():
            pltpu.sync_copy(acc.at[pl.ds(b * BR, BR)],
                            out_hbm.at[c, pl.ds(b * BR, BR)])
        return 0
    lax.fori_loop(0, KMAX, ocp, 0)


@functools.partial(
    pl.kernel,
    out_type=jax.ShapeDtypeStruct((NC, N, DW), _f32),
    mesh=_mesh,
    scratch_types=[
        pltpu.VMEM((CB,), jnp.int32),
        pltpu.VMEM((CB, DW), _f32),
        pltpu.VMEM((BR, DW), _f32),
        pltpu.VMEM_SHARED((N, DW), _f32),
        pltpu.SemaphoreType.DMA,
    ],
)
def _deg_kernel(dst_hbm, deg2_hbm, idx_v, ones_v, zb, acc, sem):
    c = lax.axis_index("c")
    s = lax.axis_index("s")
    wid = s * NC + c

    _zero_wide(zb, BR, DW)

    def fillo(i, _):
        for j in range(DW // 16):
            ones_v[i, pl.ds(j * 16, 16)] = jnp.full((16,), 1.0, _f32)
        return 0
    lax.fori_loop(0, CB, fillo, 0)

    _acc_zero(zb, acc, s)
    plsc.subcore_barrier()

    def step(t, _):
        base = wid * EW + t * CB
        pltpu.sync_copy(dst_hbm.at[pl.ds(base, CB)], idx_v)
        pltpu.sync_copy(ones_v, acc.at[idx_v], add=True)
        return 0
    lax.fori_loop(0, NCHUNK, step, 0)
    plsc.subcore_barrier()

    _acc_out(acc, deg2_hbm, c, s)


@functools.partial(
    pl.kernel,
    out_type=jax.ShapeDtypeStruct((NC, N, C), _f32),
    mesh=_mesh,
    scratch_types=[
        pltpu.VMEM((CB,), jnp.int32),
        pltpu.VMEM((CB,), jnp.int32),
        pltpu.VMEM((CB, C), _f32),
        pltpu.VMEM((BR, C), _f32),
        pltpu.VMEM_SHARED((N, C), _f32),
        pltpu.SemaphoreType.DMA,
    ],
)
def _tag_kernel(src_hbm, dst_hbm, g_hbm, hp2_hbm, isv, idv, rows, zb, acc, sem):
    c = lax.axis_index("c")
    s = lax.axis_index("s")
    wid = s * NC + c

    _zero_wide(zb, BR, C)
    _acc_zero(zb, acc, s)
    plsc.subcore_barrier()

    def step(t, _):
        base = wid * EW + t * CB
        pltpu.sync_copy(src_hbm.at[pl.ds(base, CB)], isv)
        pltpu.sync_copy(dst_hbm.at[pl.ds(base, CB)], idv)
        pltpu.async_copy(g_hbm.at[isv], rows, sem).wait()
        pltpu.sync_copy(rows, acc.at[idv], add=True)
        return 0
    lax.fori_loop(0, NCHUNK, step, 0)
    plsc.subcore_barrier()

    _acc_out(acc, hp2_hbm, c, s)


@functools.partial(
    pl.kernel,
    out_type=jax.ShapeDtypeStruct((3, G, C), _f32),
    mesh=_mesh,
    scratch_types=[
        pltpu.VMEM((CB,), jnp.int32),
        pltpu.VMEM((CB,), jnp.int32),
        pltpu.VMEM((CB, C), _f32),
        pltpu.VMEM((CB, C), _f32),
        pltpu.VMEM((CB, C), _f32),
        pltpu.VMEM((64,), jnp.int32),
        pltpu.VMEM((64, C), _f32),
        pltpu.VMEM((BR, C), _f32),
        pltpu.VMEM_SHARED((N, C), _f32),
        pltpu.SemaphoreType.DMA,
    ],
)
def _gate_kernel(src_hbm, dst_hbm, nk_hbm, nq_hbm, v_hbm, sres_hbm, roots_hbm,
                 xr3_hbm, isv, idv, kb, qb, vb, rootsv, rbuf, zb, acc, sem):
    c = lax.axis_index("c")
    s = lax.axis_index("s")
    wid = s * NC + c

    _zero_wide(zb, BR, C)
    _acc_zero(zb, acc, s)
    plsc.subcore_barrier()

    def step(t, _):
        base = wid * EW + t * CB
        pltpu.sync_copy(src_hbm.at[pl.ds(base, CB)], isv)
        pltpu.sync_copy(dst_hbm.at[pl.ds(base, CB)], idv)
        d1 = pltpu.async_copy(nk_hbm.at[idv], kb, sem)
        d2 = pltpu.async_copy(nq_hbm.at[isv], qb, sem)
        d3 = pltpu.async_copy(v_hbm.at[isv], vb, sem)
        d1.wait()
        d2.wait()
        d3.wait()

        def comp(r, _):
            for j in range(C // 16):
                sl = pl.ds(j * 16, 16)
                e = jnp.exp(kb[r, sl] + qb[r, sl])
                vb[r, sl] = vb[r, sl] / (1.0 + e)
            return 0
        lax.fori_loop(0, CB, comp, 0)

        pltpu.sync_copy(vb, acc.at[idv], add=True)
        return 0
    lax.fori_loop(0, NCHUNK, step, 0)
    plsc.subcore_barrier()

    @pl.when(s < 8)
    def _():
        pltpu.sync_copy(roots_hbm.at[pl.ds(s * 64, 64)], rootsv)
        pltpu.async_copy(acc.at[rootsv], rbuf, sem).wait()
        pltpu.sync_copy(rbuf, xr3_hbm.at[c, pl.ds(s * 64, 64)])

    @pl.when((c == 0) & (s >= 8))
    def _():
        pltpu.sync_copy(roots_hbm.at[pl.ds((s - 8) * 64, 64)], rootsv)
        pltpu.async_copy(sres_hbm.at[rootsv], rbuf, sem).wait()
        pltpu.sync_copy(rbuf, xr3_hbm.at[2, pl.ds((s - 8) * 64, 64)])


def kernel(batch_x, ts, x, last_update, edge_index, batch_vec, time_w, time_b,
           tag_w0, tag_w1, tag_b, rg_wk, rg_wq, rg_wv, rg_ws, rg_b,
           lp_w1, lp_b1, lp_w2, lp_b2):
    del last_update
    src = edge_index[0]
    dst = edge_index[1]

    h, out0 = _feat_call(batch_x, ts.reshape(N, 1), x,
                         time_w.reshape(1, TD), time_b.reshape(1, TD), tag_w0)
    deg2 = _deg_kernel(dst)
    roots = _roots_call(batch_vec.reshape(NRB, RB))
    g = _g_call(h, deg2)
    hp2 = _tag_kernel(src, dst, g)
    nk, nq, vv, sres = _h1_call(out0, hp2, deg2, tag_w1, tag_b.reshape(1, C),
                                rg_wk, rg_wq, rg_wv, rg_ws, rg_b.reshape(1, C))
    xr3 = _gate_kernel(src, dst, nk, nq, vv, sres, roots.reshape(G))
    return _mlp_call(xr3, lp_w1, lp_b1.reshape(1, C), lp_w2,
                     lp_b2.reshape(1, 1))


# speedup vs baseline: 17.8971x; 2.0925x over previous
import functools

import jax
import jax.numpy as jnp
from jax import lax
from jax.experimental import pallas as pl
from jax.experimental.pallas import tpu as pltpu
from jax.experimental.pallas import tpu_sc as plsc

N = 10000
E = 320000
G = 512
DR = 64
TD = 32
MD = 32
C = 128

NC = 2
NS = 16
NW = NC * NS
EW = E // NW
CBT = 80
NCT = EW // CBT
CBG = 40
NCG = EW // CBG
BR = 40
NBLK = N // BR
KMAX = (NBLK + NS - 1) // NS
DW = 128
RC = 32
RB = 1000
NRB = N // RB

_f32 = jnp.float32
_mesh = plsc.VectorSubcoreMesh(core_axis_name="c", subcore_axis_name="s")


def _feat_body(bx, ts, xm, tw, tb, w0, h_o, o0_o):
    rel = jnp.cos(ts[...] * tw[...] + tb[...])
    h = jnp.concatenate([bx[...], rel, xm[...]], axis=1)
    h_o[...] = h
    o0_o[...] = jnp.dot(h, w0[...], preferred_element_type=_f32)


_feat_call = pl.pallas_call(
    _feat_body,
    grid=(NRB,),
    in_specs=[
        pl.BlockSpec((RB, DR), lambda i: (i, 0)),
        pl.BlockSpec((RB, 1), lambda i: (i, 0)),
        pl.BlockSpec((RB, MD), lambda i: (i, 0)),
        pl.BlockSpec((1, TD), lambda i: (0, 0)),
        pl.BlockSpec((1, TD), lambda i: (0, 0)),
        pl.BlockSpec((C, C), lambda i: (0, 0)),
    ],
    out_specs=[
        pl.BlockSpec((RB, C), lambda i: (i, 0)),
        pl.BlockSpec((RB, C), lambda i: (i, 0)),
    ],
    out_shape=[
        jax.ShapeDtypeStruct((N, C), _f32),
        jax.ShapeDtypeStruct((N, C), _f32),
    ],
)


def _dis(deg2):
    deg = deg2[0, :, 0:1] + deg2[1, :, 0:1]
    return jnp.where(deg > 0, lax.rsqrt(jnp.maximum(deg, 1e-12)), 0.0)


def _g_body(h, deg2, g_o):
    g_o[...] = _dis(deg2[...]) * h[...]


_g_call = pl.pallas_call(
    _g_body,
    grid=(NRB,),
    in_specs=[
        pl.BlockSpec((RB, C), lambda i: (i, 0)),
        pl.BlockSpec((NC, RB, DW), lambda i: (0, i, 0)),
    ],
    out_specs=pl.BlockSpec((RB, C), lambda i: (i, 0)),
    out_shape=jax.ShapeDtypeStruct((N, C), _f32),
)


def _h1_body(o0, hp2, deg2, w1, tbias, wk, wq, wv, ws, rbias,
             nk_o, nq_o, v_o, sr_o):
    hp2v = hp2[...]
    hp = _dis(deg2[...]) * (hp2v[0] + hp2v[1])
    h1 = (o0[...] + jnp.dot(hp, w1[...], preferred_element_type=_f32)
          + tbias[...])
    nk_o[...] = -jnp.dot(h1, wk[...], preferred_element_type=_f32)
    nq_o[...] = -jnp.dot(h1, wq[...], preferred_element_type=_f32)
    v_o[...] = jnp.dot(h1, wv[...], preferred_element_type=_f32)
    sr_o[...] = jnp.dot(h1, ws[...], preferred_element_type=_f32) + rbias[...]


_h1_call = pl.pallas_call(
    _h1_body,
    grid=(NRB,),
    in_specs=[
        pl.BlockSpec((RB, C), lambda i: (i, 0)),
        pl.BlockSpec((NC, RB, C), lambda i: (0, i, 0)),
        pl.BlockSpec((NC, RB, DW), lambda i: (0, i, 0)),
        pl.BlockSpec((C, C), lambda i: (0, 0)),
        pl.BlockSpec((1, C), lambda i: (0, 0)),
        pl.BlockSpec((C, C), lambda i: (0, 0)),
        pl.BlockSpec((C, C), lambda i: (0, 0)),
        pl.BlockSpec((C, C), lambda i: (0, 0)),
        pl.BlockSpec((C, C), lambda i: (0, 0)),
        pl.BlockSpec((1, C), lambda i: (0, 0)),
    ],
    out_specs=[pl.BlockSpec((RB, C), lambda i: (i, 0)) for _ in range(4)],
    out_shape=[jax.ShapeDtypeStruct((N, C), _f32) for _ in range(4)],
)


def _roots_body(bv, r_o):
    jcol = lax.broadcasted_iota(jnp.int32, (G, 1), 0)

    def stepk(k, acc):
        row = bv[pl.ds(k, 1), :]
        cmp = (row < jcol).astype(_f32)
        return acc + jnp.sum(cmp, axis=1, keepdims=True)

    acc = lax.fori_loop(0, NRB, stepk, jnp.zeros((G, 1), _f32))
    r_o[...] = jnp.minimum(acc, float(N - 1)).astype(jnp.int32)


_roots_call = pl.pallas_call(
    _roots_body,
    in_specs=[pl.BlockSpec((NRB, RB), lambda: (0, 0))],
    out_specs=pl.BlockSpec((G, 1), lambda: (0, 0)),
    out_shape=jax.ShapeDtypeStruct((G, 1), jnp.int32),
)


def _mlp_body(xr3, w1, b1, w2, b2, o):
    xr3v = xr3[...]
    xr = xr3v[0] + xr3v[1] + xr3v[2]
    hid = jnp.maximum(jnp.dot(xr, w1[...], preferred_element_type=_f32)
                      + b1[...], 0.0)
    o[...] = jnp.dot(hid, w2[...], preferred_element_type=_f32) + b2[...]


_mlp_call = pl.pallas_call(
    _mlp_body,
    in_specs=[
        pl.BlockSpec((3, G, C), lambda: (0, 0, 0)),
        pl.BlockSpec((C, C), lambda: (0, 0)),
        pl.BlockSpec((1, C), lambda: (0, 0)),
        pl.BlockSpec((C, 1), lambda: (0, 0)),
        pl.BlockSpec((1, 1), lambda: (0, 0)),
    ],
    out_specs=pl.BlockSpec((G, 1), lambda: (0, 0)),
    out_shape=jax.ShapeDtypeStruct((G, 1), _f32),
)


def _fill_rows(buf, rows, width, value):
    def fill(i, _):
        for j in range(width // 16):
            buf[i, pl.ds(j * 16, 16)] = jnp.full((16,), value, _f32)
        return 0
    lax.fori_loop(0, rows, fill, 0)


def _fill_rows3(buf, bank, rows, width, value):
    def fill(i, _):
        for j in range(width // 16):
            buf[bank, i, pl.ds(j * 16, 16)] = jnp.full((16,), value, _f32)
        return 0
    lax.fori_loop(0, rows, fill, 0)


def _acc_zero(zb, acc, s):
    def zcp(k, _):
        b = s + k * NS

        @pl.when(b < NBLK)
        def _():
            pltpu.sync_copy(zb, acc.at[pl.ds(b * BR, BR)])
        return 0
    lax.fori_loop(0, KMAX, zcp, 0)


def _acc_out(acc, out_hbm, c, s):
    def ocp(k, _):
        b = s + k * NS

        @pl.when(b < NBLK)
        def _():
            pltpu.sync_copy(acc.at[pl.ds(b * BR, BR)],
                            out_hbm.at[c, pl.ds(b * BR, BR)])
        return 0
    lax.fori_loop(0, KMAX, ocp, 0)


def _b4(sel, fn):
    for j in range(4):
        @pl.when(sel == j)
        def _(jj=j):
            fn(jj)


def _b2(sel, fn):
    for j in range(2):
        @pl.when(sel == j)
        def _(jj=j):
            fn(jj)


@functools.partial(
    pl.kernel,
    out_type=jax.ShapeDtypeStruct((NC, N, DW), _f32),
    mesh=_mesh,
    scratch_types=[
        pltpu.VMEM((4, CBT), jnp.int32),
        pltpu.VMEM((CBT, DW), _f32),
        pltpu.VMEM((BR, DW), _f32),
        pltpu.VMEM_SHARED((N, DW), _f32),
        pltpu.SemaphoreType.DMA,
        pltpu.SemaphoreType.DMA,
    ],
)
def _deg_kernel(dst_hbm, deg2_hbm, idv, ones_v, zb, acc, isem, ssem):
    c = lax.axis_index("c")
    s = lax.axis_index("s")
    wid = s * NC + c
    base0 = wid * EW

    _fill_rows(zb, BR, DW, 0.0)
    _fill_rows(ones_v, CBT, DW, 1.0)
    _acc_zero(zb, acc, s)
    plsc.subcore_barrier()

    def load_idx(t, j):
        pltpu.async_copy(dst_hbm.at[pl.ds(base0 + t * CBT, CBT)],
                         idv.at[j], isem)

    def wait_idx():
        pltpu.make_async_copy(dst_hbm.at[pl.ds(0, CBT)], idv.at[0],
                              isem).wait()

    def wait_scat():
        pltpu.make_async_copy(deg2_hbm.at[0, pl.ds(0, CBT)], ones_v,
                              ssem).wait()

    load_idx(0, 0)
    load_idx(1, 1)
    wait_idx()

    def step(t, _):
        @pl.when(t >= 2)
        def _():
            wait_scat()

        @pl.when(t < NCT - 1)
        def _():
            wait_idx()

        _b4(t % 4, lambda j: pltpu.async_copy(
            ones_v, acc.at[idv.at[j]], ssem, add=True))

        @pl.when(t < NCT - 2)
        def _():
            _b4((t + 2) % 4, lambda j: load_idx(t + 2, j))
        return 0
    lax.fori_loop(0, NCT, step, 0)
    wait_scat()
    wait_scat()
    plsc.subcore_barrier()

    _acc_out(acc, deg2_hbm, c, s)


@functools.partial(
    pl.kernel,
    out_type=jax.ShapeDtypeStruct((NC, N, C), _f32),
    mesh=_mesh,
    scratch_types=[
        pltpu.VMEM((4, CBT), jnp.int32),
        pltpu.VMEM((4, CBT), jnp.int32),
        pltpu.VMEM((4, CBT, C), _f32),
        pltpu.VMEM_SHARED((N, C), _f32),
        pltpu.SemaphoreType.DMA,
        pltpu.SemaphoreType.DMA,
        pltpu.SemaphoreType.DMA,
    ],
)
def _tag_kernel(src_hbm, dst_hbm, g_hbm, hp2_hbm, isv, idv, rows, acc,
                isem, gsem, ssem):
    c = lax.axis_index("c")
    s = lax.axis_index("s")
    wid = s * NC + c
    base0 = wid * EW

    _fill_rows3(rows, 0, BR, C, 0.0)
    _acc_zero(rows.at[0, pl.ds(0, BR)], acc, s)
    plsc.subcore_barrier()

    def load_idx(t, j):
        pltpu.async_copy(src_hbm.at[pl.ds(base0 + t * CBT, CBT)],
                         isv.at[j], isem)
        pltpu.async_copy(dst_hbm.at[pl.ds(base0 + t * CBT, CBT)],
                         idv.at[j], isem)

    def wait_idx():
        for _ in range(2):
            pltpu.make_async_copy(src_hbm.at[pl.ds(0, CBT)], isv.at[0],
                                  isem).wait()

    def gather(j):
        pltpu.async_copy(g_hbm.at[isv.at[j]], rows.at[j], gsem)

    def wait_gather():
        pltpu.make_async_copy(g_hbm.at[pl.ds(0, CBT)], rows.at[0],
                              gsem).wait()

    def wait_scat():
        pltpu.make_async_copy(g_hbm.at[pl.ds(0, CBT)], rows.at[0],
                              ssem).wait()

    load_idx(0, 0)
    load_idx(1, 1)
    wait_idx()
    gather(0)

    def step(t, _):
        @pl.when(t >= 2)
        def _():
            wait_scat()

        @pl.when(t < NCT - 1)
        def _():
            wait_idx()
            _b4((t + 1) % 4, gather)

        @pl.when(t < NCT - 2)
        def _():
            _b4((t + 2) % 4, lambda j: load_idx(t + 2, j))

        wait_gather()
        _b4(t % 4, lambda j: pltpu.async_copy(
            rows.at[j], acc.at[idv.at[j]], ssem, add=True))
        return 0
    lax.fori_loop(0, NCT, step, 0)
    wait_scat()
    wait_scat()
    plsc.subcore_barrier()

    _acc_out(acc, hp2_hbm, c, s)


@functools.partial(
    pl.kernel,
    out_type=jax.ShapeDtypeStruct((3, G, C), _f32),
    mesh=_mesh,
    scratch_types=[
        pltpu.VMEM((4, CBG), jnp.int32),
        pltpu.VMEM((4, CBG), jnp.int32),
        pltpu.VMEM((2, CBG, C), _f32),
        pltpu.VMEM((2, CBG, C), _f32),
        pltpu.VMEM((2, CBG, C), _f32),
        pltpu.VMEM((2, CBG, C), _f32),
        pltpu.VMEM((RC,), jnp.int32),
        pltpu.VMEM((RC, C), _f32),
        pltpu.VMEM_SHARED((N, C), _f32),
        pltpu.SemaphoreType.DMA,
        pltpu.SemaphoreType.DMA,
        pltpu.SemaphoreType.DMA,
    ],
)
def _gate_kernel(src_hbm, dst_hbm, nk_hbm, nq_hbm, v_hbm, sres_hbm, roots_hbm,
                 xr3_hbm, isv, idv, kb, qb, vb, ob, rootsv, rbuf, acc,
                 isem, gsem, ssem):
    c = lax.axis_index("c")
    s = lax.axis_index("s")
    wid = s * NC + c
    base0 = wid * EW

    _fill_rows3(ob, 0, BR, C, 0.0)
    _acc_zero(ob.at[0, pl.ds(0, BR)], acc, s)
    plsc.subcore_barrier()

    def load_idx(t, j):
        pltpu.async_copy(src_hbm.at[pl.ds(base0 + t * CBG, CBG)],
                         isv.at[j], isem)
        pltpu.async_copy(dst_hbm.at[pl.ds(base0 + t * CBG, CBG)],
                         idv.at[j], isem)

    def wait_idx():
        for _ in range(2):
            pltpu.make_async_copy(src_hbm.at[pl.ds(0, CBG)], isv.at[0],
                                  isem).wait()

    def gather(j):
        b = j % 2
        pltpu.async_copy(nk_hbm.at[idv.at[j]], kb.at[b], gsem)
        pltpu.async_copy(nq_hbm.at[isv.at[j]], qb.at[b], gsem)
        pltpu.async_copy(v_hbm.at[isv.at[j]], vb.at[b], gsem)

    def wait_gather():
        for _ in range(3):
            pltpu.make_async_copy(nk_hbm.at[pl.ds(0, CBG)], kb.at[0],
                                  gsem).wait()

    def wait_scat():
        pltpu.make_async_copy(nk_hbm.at[pl.ds(0, CBG)], ob.at[0],
                              ssem).wait()

    def compute(b):
        def crow(r, _):
            for j8 in range(C // 16):
                sl = pl.ds(j8 * 16, 16)
                e = jnp.exp(kb[b, r, sl] + qb[b, r, sl])
                ob[b, r, sl] = vb[b, r, sl] / (1.0 + e)
            return 0
        lax.fori_loop(0, CBG, crow, 0)

    load_idx(0, 0)
    load_idx(1, 1)
    wait_idx()
    gather(0)

    def step(t, _):
        @pl.when(t >= 2)
        def _():
            wait_scat()

        @pl.when(t < NCG - 1)
        def _():
            wait_idx()
            _b4((t + 1) % 4, gather)

        @pl.when(t < NCG - 2)
        def _():
            _b4((t + 2) % 4, lambda j: load_idx(t + 2, j))

        wait_gather()
        _b2(t % 2, compute)
        _b4(t % 4, lambda j: pltpu.async_copy(
            ob.at[j % 2], acc.at[idv.at[j]], ssem, add=True))
        return 0
    lax.fori_loop(0, NCG, step, 0)
    wait_scat()
    wait_scat()
    plsc.subcore_barrier()

    pltpu.sync_copy(roots_hbm.at[pl.ds(s * RC, RC)], rootsv)
    pltpu.async_copy(acc.at[rootsv], rbuf, gsem).wait()
    pltpu.sync_copy(rbuf, xr3_hbm.at[c, pl.ds(s * RC, RC)])

    @pl.when(c == 0)
    def _():
        pltpu.async_copy(sres_hbm.at[rootsv], rbuf, gsem).wait()
        pltpu.sync_copy(rbuf, xr3_hbm.at[2, pl.ds(s * RC, RC)])


def kernel(batch_x, ts, x, last_update, edge_index, batch_vec, time_w, time_b,
           tag_w0, tag_w1, tag_b, rg_wk, rg_wq, rg_wv, rg_ws, rg_b,
           lp_w1, lp_b1, lp_w2, lp_b2):
    del last_update
    src = edge_index[0]
    dst = edge_index[1]

    h, out0 = _feat_call(batch_x, ts.reshape(N, 1), x,
                         time_w.reshape(1, TD), time_b.reshape(1, TD), tag_w0)
    deg2 = _deg_kernel(dst)
    roots = _roots_call(batch_vec.reshape(NRB, RB))
    g = _g_call(h, deg2)
    hp2 = _tag_kernel(src, dst, g)
    nk, nq, vv, sres = _h1_call(out0, hp2, deg2, tag_w1, tag_b.reshape(1, C),
                                rg_wk, rg_wq, rg_wv, rg_ws, rg_b.reshape(1, C))
    xr3 = _gate_kernel(src, dst, nk, nq, vv, sres, roots.reshape(G))
    return _mlp_call(xr3, lp_w1, lp_b1.reshape(1, C), lp_w2,
                     lp_b2.reshape(1, 1))
